# Initial kernel scaffold; baseline (speedup 1.0000x reference)
import jax
import jax.numpy as jnp
from jax.experimental import pallas as pl


def kernel(x, edge_src1, edge_dst1, edge_val1, edge_src2, edge_dst2, edge_val2, weight, biases):
    raise NotImplementedError("write your pallas kernel here")


# speedup vs baseline: 1.0150x; 1.0150x over previous
import jax
import jax.numpy as jnp
from jax.experimental import pallas as pl
from jax.experimental.pallas import tpu as pltpu

N = 10000
D = 128
B = 8
OUT = 128
NUM_MAT = 5
N_BLK = 2000


def _spmm(src, dst, val, m):
    return jax.ops.segment_sum(val[:, None] * jnp.take(m, src, axis=0), dst,
                               num_segments=N)


def _matmul_body(x0_ref, s11_ref, s21_ref, s12_ref, s22_ref, w_ref, b_ref,
                 o_ref):
    acc = jnp.dot(x0_ref[0], w_ref[0], preferred_element_type=jnp.float32)
    acc += jnp.dot(s11_ref[0], w_ref[1], preferred_element_type=jnp.float32)
    acc += jnp.dot(s21_ref[0], w_ref[2], preferred_element_type=jnp.float32)
    acc += jnp.dot(s12_ref[0], w_ref[3], preferred_element_type=jnp.float32)
    acc += jnp.dot(s22_ref[0], w_ref[4], preferred_element_type=jnp.float32)
    o_ref[0] = acc + b_ref[...]


def _final_matmul(x0, s11, s21, s12, s22, wf, biases):
    in_spec = pl.BlockSpec((1, N_BLK, D), lambda b, n: (b, n, 0))
    return pl.pallas_call(
        _matmul_body,
        grid=(B, N // N_BLK),
        in_specs=[in_spec] * 5 + [
            pl.BlockSpec((NUM_MAT, D, OUT), lambda b, n: (0, 0, 0)),
            pl.BlockSpec((OUT,), lambda b, n: (0,)),
        ],
        out_specs=pl.BlockSpec((1, N_BLK, OUT), lambda b, n: (b, n, 0)),
        out_shape=jax.ShapeDtypeStruct((B, N, OUT), jnp.float32),
    )(x0, s11, s21, s12, s22, wf, biases)


def kernel(x, edge_src1, edge_dst1, edge_val1, edge_src2, edge_dst2,
           edge_val2, weight, biases):
    b, n, d = x.shape
    x0 = jnp.transpose(x, (1, 0, 2)).reshape(n, b * d)

    s11 = _spmm(edge_src1, edge_dst1, edge_val1, x0)
    s21 = _spmm(edge_src1, edge_dst1, edge_val1, s11)
    s12 = _spmm(edge_src2, edge_dst2, edge_val2, x0)
    s22 = _spmm(edge_src2, edge_dst2, edge_val2, s12)

    def cm(v):
        return jnp.transpose(v.reshape(n, b, d), (1, 0, 2))

    w = weight.reshape(d, NUM_MAT, OUT)
    wf = jnp.stack([
        w[:, 0] - w[:, 2] - w[:, 4],
        w[:, 1],
        2.0 * w[:, 2],
        w[:, 3],
        2.0 * w[:, 4],
    ])
    return _final_matmul(x, cm(s11), cm(s21), cm(s12), cm(s22), wf, biases)


# speedup vs baseline: 1.4214x; 1.4004x over previous
import functools

import jax
import jax.numpy as jnp
from jax import lax
from jax.experimental import pallas as pl
from jax.experimental.pallas import tpu as pltpu
from jax.experimental.pallas import tpu_sc as plsc

N = 10000
D = 128
B = 8
OUT = 128
NUM_MAT = 5
N_BLK = 2000

NC = 2
NS = 16
W = 128
E = 320000
NWIN = 157
EPS = NWIN * W
EPAD = EPS * NS
CHUNKS_PER_SC = B // NC
RS = 624
R_REM = N - NS * RS
ZCHUNKS = (128, 128, 128, 128, 112)


_GATHER_DNUMS = lax.GatherDimensionNumbers(
    offset_dims=(), collapsed_slice_dims=(0,), start_index_map=(0,))


def _bcast_lane(vec16, e):
    idx = jnp.full((16, 1), e, jnp.int32)
    return lax.gather(vec16, idx, _GATHER_DNUMS, slice_sizes=(1,),
                      mode=lax.GatherScatterMode.PROMISE_IN_BOUNDS)


def _spmm_body(x_hbm, src_hbm, dst_hbm, val_hbm, out_hbm,
               acc_sh, src_v, dst_v, val_v, adj_v, rows_v, scaled_v,
               zero_v, sem):
    cid = lax.axis_index("c")
    sid = lax.axis_index("s")
    ebase = sid * EPS

    @pl.loop(0, 128)
    def _(r):
        for q in range(8):
            zero_v[r, pl.ds(q * 16, 16)] = jnp.zeros((16,), jnp.float32)

    @pl.loop(0, CHUNKS_PER_SC)
    def _(bi):
        bb = cid * CHUNKS_PER_SC + bi
        row_off = bb * N

        off = 0
        for zc in ZCHUNKS:
            pltpu.sync_copy(zero_v.at[pl.ds(0, zc)],
                            acc_sh.at[pl.ds(sid * RS + off, zc)])
            off += zc

        @pl.when(sid == 0)
        def _():
            pltpu.sync_copy(zero_v.at[pl.ds(0, R_REM)],
                            acc_sh.at[pl.ds(NS * RS, R_REM)])
        plsc.subcore_barrier()

        @pl.loop(0, NWIN)
        def _(w):
            base = ebase + w * W
            pltpu.sync_copy(src_hbm.at[pl.ds(base, W)], src_v)
            pltpu.sync_copy(dst_hbm.at[pl.ds(base, W)], dst_v)
            pltpu.sync_copy(val_hbm.at[pl.ds(base, W)], val_v)

            @pl.loop(0, W // 16)
            def _(g):
                adj_v[pl.ds(g * 16, 16)] = (
                    src_v[pl.ds(g * 16, 16)] + row_off)

            pltpu.async_copy(x_hbm.at[adj_v], rows_v, sem).wait()

            @pl.loop(0, W // 16)
            def _(g):
                v16 = val_v[pl.ds(g * 16, 16)]
                for e in range(16):
                    bc = _bcast_lane(v16, e)
                    r = g * 16 + e
                    for q in range(8):
                        sl = pl.ds(q * 16, 16)
                        scaled_v[r, sl] = rows_v[r, sl] * bc

            pltpu.sync_copy(scaled_v, acc_sh.at[dst_v], add=True)

        plsc.subcore_barrier()
        pltpu.sync_copy(acc_sh.at[pl.ds(sid * RS, RS)],
                        out_hbm.at[bb].at[pl.ds(sid * RS, RS)])

        @pl.when(sid == 0)
        def _():
            pltpu.sync_copy(acc_sh.at[pl.ds(NS * RS, R_REM)],
                            out_hbm.at[bb].at[pl.ds(NS * RS, R_REM)])
        plsc.subcore_barrier()


@jax.jit
def _spmm_sc(x_cm, src, dst, val):
    mesh = plsc.VectorSubcoreMesh(core_axis_name="c", subcore_axis_name="s")
    x_flat = x_cm.reshape(B * N, D)
    kern = pl.kernel(
        _spmm_body,
        out_type=jax.ShapeDtypeStruct((B, N, D), jnp.float32),
        mesh=mesh,
        scratch_types=[
            pltpu.VMEM_SHARED((N, D), jnp.float32),
            pltpu.VMEM((W,), jnp.int32),
            pltpu.VMEM((W,), jnp.int32),
            pltpu.VMEM((W,), jnp.float32),
            pltpu.VMEM((W,), jnp.int32),
            pltpu.VMEM((W, D), jnp.float32),
            pltpu.VMEM((W, D), jnp.float32),
            pltpu.VMEM((128, D), jnp.float32),
            pltpu.SemaphoreType.DMA,
        ],
    )
    return kern(x_flat, src, dst, val)


def _matmul_body(x0_ref, s11_ref, s21_ref, s12_ref, s22_ref, w_ref, b_ref,
                 o_ref):
    acc = jnp.dot(x0_ref[0], w_ref[0], preferred_element_type=jnp.float32)
    acc += jnp.dot(s11_ref[0], w_ref[1], preferred_element_type=jnp.float32)
    acc += jnp.dot(s21_ref[0], w_ref[2], preferred_element_type=jnp.float32)
    acc += jnp.dot(s12_ref[0], w_ref[3], preferred_element_type=jnp.float32)
    acc += jnp.dot(s22_ref[0], w_ref[4], preferred_element_type=jnp.float32)
    o_ref[0] = acc + b_ref[...]


def _final_matmul(x0, s11, s21, s12, s22, wf, biases):
    in_spec = pl.BlockSpec((1, N_BLK, D), lambda b, n: (b, n, 0))
    return pl.pallas_call(
        _matmul_body,
        grid=(B, N // N_BLK),
        in_specs=[in_spec] * 5 + [
            pl.BlockSpec((NUM_MAT, D, OUT), lambda b, n: (0, 0, 0)),
            pl.BlockSpec((OUT,), lambda b, n: (0,)),
        ],
        out_specs=pl.BlockSpec((1, N_BLK, OUT), lambda b, n: (b, n, 0)),
        out_shape=jax.ShapeDtypeStruct((B, N, OUT), jnp.float32),
    )(x0, s11, s21, s12, s22, wf, biases)


def _pad_edges(src, dst, val):
    pad = EPAD - E
    return (jnp.pad(src, (0, pad)), jnp.pad(dst, (0, pad)),
            jnp.pad(val, (0, pad)))


def kernel(x, edge_src1, edge_dst1, edge_val1, edge_src2, edge_dst2,
           edge_val2, weight, biases):
    b, n, d = x.shape
    s1, d1, v1 = _pad_edges(edge_src1, edge_dst1, edge_val1)
    s2, d2, v2 = _pad_edges(edge_src2, edge_dst2, edge_val2)

    s11 = _spmm_sc(x, s1, d1, v1)
    s21 = _spmm_sc(s11, s1, d1, v1)
    s12 = _spmm_sc(x, s2, d2, v2)
    s22 = _spmm_sc(s12, s2, d2, v2)

    w = weight.reshape(d, NUM_MAT, OUT)
    wf = jnp.stack([
        w[:, 0] - w[:, 2] - w[:, 4],
        w[:, 1],
        2.0 * w[:, 2],
        w[:, 3],
        2.0 * w[:, 4],
    ])
    return _final_matmul(x, s11, s21, s12, s22, wf, biases)
